# baseline (speedup 1.0000x reference)
import functools

import jax
import jax.numpy as jnp
from jax.experimental import pallas as pl

N = 64
ROWS = 16 * 512


def _band_mask():
    i = jax.lax.broadcasted_iota(jnp.int32, (N, N), 0)
    j = jax.lax.broadcasted_iota(jnp.int32, (N, N), 1)
    m = j - i
    g1 = (m >= 0) & (m <= 15)
    g2 = (m >= 17) & (m <= 31) & (m % 2 == 1) & (i % 2 == 0)
    g3 = (m >= 35) & (m <= 63) & (m % 4 == 3) & (i % 4 == 0)
    return g1 | g2 | g3


def _pool_kernel(x_ref, o_ref):
    xb = x_ref[...]
    R = xb.shape[0]
    neg = jnp.float32(-jnp.inf)
    i = jax.lax.broadcasted_iota(jnp.int32, (N, N), 0)
    j = jax.lax.broadcasted_iota(jnp.int32, (N, N), 1)
    lower = (j >= i)[None, :, :]
    t = jnp.where(lower, xb[:, None, :], neg)
    s = 1
    while s < N:
        pad = jnp.full((R, N, s), neg, jnp.float32)
        shifted = jnp.concatenate([pad, t[:, :, : N - s]], axis=-1)
        t = jnp.maximum(t, shifted)
        s *= 2
    mask = _band_mask()[None, :, :]
    o_ref[...] = jnp.where(mask, t, jnp.float32(0.0))


@functools.partial(jax.jit, static_argnames=("rows_per_block",))
def _run(x2d, rows_per_block):
    grid = (ROWS // rows_per_block,)
    return pl.pallas_call(
        _pool_kernel,
        grid=grid,
        in_specs=[pl.BlockSpec((rows_per_block, N), lambda r: (r, 0))],
        out_specs=pl.BlockSpec((rows_per_block, N, N), lambda r: (r, 0, 0)),
        out_shape=jax.ShapeDtypeStruct((ROWS, N, N), jnp.float32),
    )(x2d)


def kernel(x):
    B, D, n = x.shape
    x2d = x.reshape(B * D, n)
    out = _run(x2d, 256)
    return out.reshape(B, D, n, n)


# speedup vs baseline: 1.0096x; 1.0096x over previous
import functools

import jax
import jax.numpy as jnp
from jax.experimental import pallas as pl

N = 64
ROWS = 16 * 512


def _band_mask():
    i = jax.lax.broadcasted_iota(jnp.int32, (N, N), 0)
    j = jax.lax.broadcasted_iota(jnp.int32, (N, N), 1)
    m = j - i
    g1 = (m >= 0) & (m <= 15)
    g2 = (m >= 17) & (m <= 31) & (m % 2 == 1) & (i % 2 == 0)
    g3 = (m >= 35) & (m <= 63) & (m % 4 == 3) & (i % 4 == 0)
    return g1 | g2 | g3


def _pool_kernel(x_ref, o_ref):
    xb = x_ref[...]
    R = xb.shape[0]
    neg = jnp.float32(-jnp.inf)
    x2 = jnp.concatenate([xb, xb], axis=-1)
    p = jax.lax.broadcasted_iota(jnp.int32, (N // 2, 2 * N), 0)
    l = jax.lax.broadcasted_iota(jnp.int32, (N // 2, 2 * N), 1)
    i = 2 * p + l // N
    j = l % N
    t = jnp.where((j >= i)[None], x2[:, None, :], neg)
    s = 1
    while s < N:
        pad = jnp.full((R, N // 2, s), neg, jnp.float32)
        shifted = jnp.concatenate([pad, t[:, :, : 2 * N - s]], axis=-1)
        t = jnp.maximum(t, jnp.where((j >= s)[None], shifted, neg))
        s *= 2
    m = j - i
    g1 = (m >= 0) & (m <= 15)
    g2 = (m >= 17) & (m <= 31) & (m % 2 == 1) & (i % 2 == 0)
    g3 = (m >= 35) & (m <= 63) & (m % 4 == 3) & (i % 4 == 0)
    mask = (g1 | g2 | g3)[None]
    o_ref[...] = jnp.where(mask, t, jnp.float32(0.0))


@functools.partial(jax.jit, static_argnames=("rows_per_block",))
def _run(x2d, rows_per_block):
    grid = (ROWS // rows_per_block,)
    return pl.pallas_call(
        _pool_kernel,
        grid=grid,
        in_specs=[pl.BlockSpec((rows_per_block, N), lambda r: (r, 0))],
        out_specs=pl.BlockSpec(
            (rows_per_block, N // 2, 2 * N), lambda r: (r, 0, 0)
        ),
        out_shape=jax.ShapeDtypeStruct((ROWS, N // 2, 2 * N), jnp.float32),
    )(x2d)


def kernel(x):
    B, D, n = x.shape
    x2d = x.reshape(B * D, n)
    out = _run(x2d, 256)
    return out.reshape(B, D, n, n)
